# baseline (speedup 1.0000x reference)
import functools

import jax
import jax.numpy as jnp
from jax import lax
from jax.experimental import pallas as pl
from jax.experimental.pallas import tpu as pltpu, tpu_sc as plsc

_NC = 2
_NS = 16
_LANE = 16
_TR = 8
_BW = 2048


def _sc_body(B, L, D, Vp, tokT_hbm, tabT_hbm, out_t,
             tabT_v, trow_v, obuf_a, obuf_b, sem_a, sem_b):
    c = lax.axis_index("c")
    s = lax.axis_index("s")
    wid = s * _NC + c
    nhalf = B // _BW
    units_total = L * D // _TR * nhalf
    n_units = units_total // (_NC * _NS)
    u0 = wid * n_units

    pltpu.sync_copy(tabT_hbm.at[:, :], tabT_v)

    ng = _BW // _LANE

    def splat(x):
        return jnp.full((_LANE,), x, jnp.int32)

    def compute(tr, obuf):
        d0 = tr * _TR
        dvecs = [splat(d0 + d) for d in range(_TR)]

        @plsc.parallel_loop(0, ng, 1, unroll=4)
        def gloop(g):
            tokv = trow_v[pl.ds(g * _LANE, _LANE)]
            for d in range(_TR):
                v = plsc.load_gather(tabT_v, [dvecs[d], tokv])
                obuf[d, pl.ds(g * _LANE, _LANE)] = v

    def drain(obuf, sem):
        pltpu.make_async_copy(
            obuf, out_t.at[0, pl.ds(0, _TR), pl.ds(0, _BW)], sem).wait()

    bufs = [(obuf_a, sem_a), (obuf_b, sem_b)]

    def ubody(j, carry):
        u = u0 + j
        l = u // (nhalf * (D // _TR))
        rem = u % (nhalf * (D // _TR))
        half = rem // (D // _TR)
        tr = rem % (D // _TR)

        pl.when(jnp.logical_or(j == 0, tr == 0))(
            lambda: pltpu.sync_copy(
                tokT_hbm.at[l, pl.ds(half * _BW, _BW)], trow_v))

        for i, (ob, sm) in enumerate(bufs):
            @pl.when(j % 2 == i)
            def _(ob=ob, sm=sm):
                pl.when(j >= 2)(lambda: drain(ob, sm))
                compute(tr, ob)
                pltpu.async_copy(
                    ob, out_t.at[l, pl.ds(tr * _TR, _TR), pl.ds(half * _BW, _BW)],
                    sm)
        return carry

    lax.fori_loop(0, n_units, ubody, 0)
    for ob, sm in bufs:
        drain(ob, sm)


def kernel(tokens, embedding, embedding2):
    B, L = tokens.shape
    V, D = embedding.shape
    assert B % _BW == 0 and D % _TR == 0
    assert (L * (D // _TR) * (B // _BW)) % (_NC * _NS) == 0

    tokT = tokens.astype(jnp.int32).T
    Vp = 1024
    pad = ((0, 0), (0, Vp - V))

    mesh = plsc.VectorSubcoreMesh(core_axis_name="c", subcore_axis_name="s")
    run = pl.kernel(
        functools.partial(_sc_body, B, L, D, Vp),
        mesh=mesh,
        out_type=[jax.ShapeDtypeStruct((L, D, B), jnp.float32)],
        scratch_types=[
            pltpu.VMEM((D, Vp), jnp.float32),
            pltpu.VMEM((_BW,), jnp.int32),
            pltpu.VMEM((_TR, _BW), jnp.float32),
            pltpu.VMEM((_TR, _BW), jnp.float32),
            pltpu.SemaphoreType.DMA,
            pltpu.SemaphoreType.DMA,
        ],
        compiler_params=pltpu.CompilerParams(
            needs_layout_passes=False, disable_bounds_checks=True),
    )
    (o1,) = run(tokT, jnp.pad(embedding.T, pad))
    (o2,) = run(tokT, jnp.pad(embedding2.T, pad))
    return (o1.transpose(2, 0, 1), o2.transpose(2, 0, 1))


# speedup vs baseline: 1.0133x; 1.0133x over previous
import functools

import jax
import jax.numpy as jnp
from jax import lax
from jax.experimental import pallas as pl
from jax.experimental.pallas import tpu as pltpu, tpu_sc as plsc

_NC = 2
_NS = 16
_LANE = 16
_TR = 8
_BW = 2048


def _sc_body(B, L, D, Vp, tokT_hbm, tabT_hbm, out_t,
             tabT_v, trow_v, obuf_a, obuf_b, sem_a, sem_b):
    c = lax.axis_index("c")
    s = lax.axis_index("s")
    wid = s * _NC + c
    nhalf = B // _BW
    units_total = L * D // _TR * nhalf
    n_units = units_total // (_NC * _NS)
    u0 = wid * n_units

    pltpu.sync_copy(tabT_hbm.at[:, :], tabT_v)

    ng = _BW // _LANE

    def splat(x):
        return jnp.full((_LANE,), x, jnp.int32)

    def compute(tr, obuf):
        d0 = tr * _TR
        dvecs = [splat(d0 + d) for d in range(_TR)]

        @plsc.parallel_loop(0, ng, 1, unroll=2)
        def gloop(g):
            tokv = trow_v[pl.ds(g * _LANE, _LANE)]
            for d in range(_TR):
                v = plsc.load_gather(tabT_v, [dvecs[d], tokv])
                obuf[d, pl.ds(g * _LANE, _LANE)] = v

    def drain(obuf, sem):
        pltpu.make_async_copy(
            obuf, out_t.at[0, pl.ds(0, _TR), pl.ds(0, _BW)], sem).wait()

    bufs = [(obuf_a, sem_a), (obuf_b, sem_b)]

    def ubody(j, carry):
        u = u0 + j
        l = u // (nhalf * (D // _TR))
        rem = u % (nhalf * (D // _TR))
        half = rem // (D // _TR)
        tr = rem % (D // _TR)

        pl.when(jnp.logical_or(j == 0, tr == 0))(
            lambda: pltpu.sync_copy(
                tokT_hbm.at[l, pl.ds(half * _BW, _BW)], trow_v))

        for i, (ob, sm) in enumerate(bufs):
            @pl.when(j % 2 == i)
            def _(ob=ob, sm=sm):
                pl.when(j >= 2)(lambda: drain(ob, sm))
                compute(tr, ob)
                pltpu.async_copy(
                    ob, out_t.at[l, pl.ds(tr * _TR, _TR), pl.ds(half * _BW, _BW)],
                    sm)
        return carry

    lax.fori_loop(0, n_units, ubody, 0)
    for ob, sm in bufs:
        drain(ob, sm)


def kernel(tokens, embedding, embedding2):
    B, L = tokens.shape
    V, D = embedding.shape
    assert B % _BW == 0 and D % _TR == 0
    assert (L * (D // _TR) * (B // _BW)) % (_NC * _NS) == 0

    tokT = tokens.astype(jnp.int32).T
    Vp = 1024
    pad = ((0, 0), (0, Vp - V))

    mesh = plsc.VectorSubcoreMesh(core_axis_name="c", subcore_axis_name="s")
    run = pl.kernel(
        functools.partial(_sc_body, B, L, D, Vp),
        mesh=mesh,
        out_type=[jax.ShapeDtypeStruct((L, D, B), jnp.float32)],
        scratch_types=[
            pltpu.VMEM((D, Vp), jnp.float32),
            pltpu.VMEM((_BW,), jnp.int32),
            pltpu.VMEM((_TR, _BW), jnp.float32),
            pltpu.VMEM((_TR, _BW), jnp.float32),
            pltpu.SemaphoreType.DMA,
            pltpu.SemaphoreType.DMA,
        ],
        compiler_params=pltpu.CompilerParams(
            needs_layout_passes=False, disable_bounds_checks=True),
    )
    (o1,) = run(tokT, jnp.pad(embedding.T, pad))
    (o2,) = run(tokT, jnp.pad(embedding2.T, pad))
    return (o1.transpose(2, 0, 1), o2.transpose(2, 0, 1))


# speedup vs baseline: 1.0597x; 1.0457x over previous
import functools

import jax
import jax.numpy as jnp
from jax import lax
from jax.experimental import pallas as pl
from jax.experimental.pallas import tpu as pltpu, tpu_sc as plsc

_NC = 2
_NS = 16
_LANE = 16
_TR = 8
_BW = 2048


def _sc_body(B, L, D, Vp, tokT_hbm, tabT_hbm, out_t,
             tabT_v, trow_v, obuf_a, obuf_b, sem_a, sem_b):
    c = lax.axis_index("c")
    s = lax.axis_index("s")
    wid = s * _NC + c
    nhalf = B // _BW
    units_total = L * D // _TR * nhalf
    n_units = units_total // (_NC * _NS)
    u0 = wid * n_units

    pltpu.sync_copy(tabT_hbm.at[:, :], tabT_v)

    ng = _BW // _LANE

    def splat(x):
        return jnp.full((_LANE,), x, jnp.int32)

    def compute(tr, hoff, obuf):
        d0 = tr * _TR
        dvecs = [splat(d0 + d) for d in range(_TR)]

        @plsc.parallel_loop(0, ng, 1, unroll=2)
        def gloop(g):
            tokv = trow_v[pl.ds(hoff + g * _LANE, _LANE)]
            for d in range(_TR):
                v = plsc.load_gather(tabT_v, [dvecs[d], tokv])
                obuf[d, pl.ds(g * _LANE, _LANE)] = v

    def drain(obuf, sem):
        pltpu.make_async_copy(
            obuf, out_t.at[0, pl.ds(0, _TR), pl.ds(0, _BW)], sem).wait()

    bufs = [(obuf_a, sem_a), (obuf_b, sem_b)]

    def ubody(j, carry):
        u = u0 + j
        l = u // (nhalf * (D // _TR))
        rem = u % (nhalf * (D // _TR))
        half = rem // (D // _TR)
        tr = rem % (D // _TR)

        pl.when(jnp.logical_or(j == 0, rem == 0))(
            lambda: pltpu.sync_copy(tokT_hbm.at[l, :], trow_v))
        hoff = half * _BW

        for i, (ob, sm) in enumerate(bufs):
            @pl.when(j % 2 == i)
            def _(ob=ob, sm=sm):
                pl.when(j >= 2)(lambda: drain(ob, sm))
                compute(tr, hoff, ob)
                pltpu.async_copy(
                    ob, out_t.at[l, pl.ds(tr * _TR, _TR), pl.ds(half * _BW, _BW)],
                    sm)
        return carry

    lax.fori_loop(0, n_units, ubody, 0)
    for ob, sm in bufs:
        drain(ob, sm)


def kernel(tokens, embedding, embedding2):
    B, L = tokens.shape
    V, D = embedding.shape
    assert B % _BW == 0 and D % _TR == 0
    assert (L * (D // _TR) * (B // _BW)) % (_NC * _NS) == 0

    tokT = tokens.astype(jnp.int32).T
    Vp = 1024
    pad = ((0, 0), (0, Vp - V))

    mesh = plsc.VectorSubcoreMesh(core_axis_name="c", subcore_axis_name="s")
    run = pl.kernel(
        functools.partial(_sc_body, B, L, D, Vp),
        mesh=mesh,
        out_type=[jax.ShapeDtypeStruct((L, D, B), jnp.float32)],
        scratch_types=[
            pltpu.VMEM((D, Vp), jnp.float32),
            pltpu.VMEM((B,), jnp.int32),
            pltpu.VMEM((_TR, _BW), jnp.float32),
            pltpu.VMEM((_TR, _BW), jnp.float32),
            pltpu.SemaphoreType.DMA,
            pltpu.SemaphoreType.DMA,
        ],
        compiler_params=pltpu.CompilerParams(
            needs_layout_passes=False, disable_bounds_checks=True),
    )
    (o1,) = run(tokT, jnp.pad(embedding.T, pad))
    (o2,) = run(tokT, jnp.pad(embedding2.T, pad))
    return (o1.transpose(2, 0, 1), o2.transpose(2, 0, 1))
